# baseline (speedup 1.0000x reference)
import functools

import jax
import jax.numpy as jnp
from jax import lax
from jax.experimental import pallas as pl
from jax.experimental.pallas import tpu as pltpu
from jax.experimental.pallas import tpu_sc as plsc

_T, _D, _H, _E, _O, _K = 2048, 1024, 1024, 8, 1024, 2

_BB = 256
_L = _T * _K + _E * _BB
_NBLK = _L // _BB

_NC, _NS = 2, 16
_NW = _NC * _NS
_GCH = 64


def _gate_body(x_ref, gw_ref, gb_ref, gow_ref, gob_ref,
               i0_ref, i1_ref, g0_ref, g1_ref):
    x = x_ref[...]
    h = jnp.maximum(
        jnp.dot(x, gw_ref[...], preferred_element_type=jnp.float32)
        + gb_ref[...], 0.0)
    logits = (jnp.dot(h, gow_ref[...], preferred_element_type=jnp.float32)
              + gob_ref[...])
    cols = lax.broadcasted_iota(jnp.int32, logits.shape, 1)
    v0 = jnp.max(logits, axis=1, keepdims=True)
    i0 = jnp.min(jnp.where(logits == v0, cols, _E), axis=1, keepdims=True)
    masked = jnp.where(cols == i0, -jnp.inf, logits)
    v1 = jnp.max(masked, axis=1, keepdims=True)
    i1 = jnp.min(jnp.where(masked == v1, cols, _E), axis=1, keepdims=True)
    e1 = jnp.exp(v1 - v0)
    g0 = 1.0 / (1.0 + e1)
    i0_ref[...] = i0
    i1_ref[...] = i1
    g0_ref[...] = g0
    g1_ref[...] = 1.0 - g0


def _gate_topk(x, gate_w, gate_b, gate_out_w, gate_out_b):
    tb = 512
    grid = (_T // tb,)
    out_shape = [
        jax.ShapeDtypeStruct((_T, 1), jnp.int32),
        jax.ShapeDtypeStruct((_T, 1), jnp.int32),
        jax.ShapeDtypeStruct((_T, 1), jnp.float32),
        jax.ShapeDtypeStruct((_T, 1), jnp.float32),
    ]
    tspec = lambda: pl.BlockSpec((tb, 1), lambda i: (i, 0))
    return pl.pallas_call(
        _gate_body,
        grid=grid,
        in_specs=[
            pl.BlockSpec((tb, _D), lambda i: (i, 0)),
            pl.BlockSpec((_D, _H), lambda i: (0, 0)),
            pl.BlockSpec((1, _H), lambda i: (0, 0)),
            pl.BlockSpec((_H, _E), lambda i: (0, 0)),
            pl.BlockSpec((1, _E), lambda i: (0, 0)),
        ],
        out_specs=[tspec(), tspec(), tspec(), tspec()],
        out_shape=out_shape,
        compiler_params=pltpu.CompilerParams(
            dimension_semantics=("arbitrary",)),
    )(x, gate_w, gate_b.reshape(1, _H), gate_out_w, gate_out_b.reshape(1, _E))


def _sc_gather_body(rpw, x_hbm, idx_hbm, out_hbm, idx_v, rows_v, sem):
    wid = lax.axis_index("s") * _NC + lax.axis_index("c")
    base = wid * rpw
    pltpu.sync_copy(idx_hbm.at[pl.ds(base, rpw)], idx_v)
    for c in range(rpw // _GCH):
        pltpu.async_copy(
            x_hbm.at[idx_v.at[pl.ds(c * _GCH, _GCH)]], rows_v, sem).wait()
        pltpu.sync_copy(rows_v, out_hbm.at[pl.ds(base + c * _GCH, _GCH)])


def _sc_gather(x, idx):
    n = idx.shape[0]
    rpw = n // _NW
    mesh = plsc.VectorSubcoreMesh(core_axis_name="c", subcore_axis_name="s",
                                  num_cores=_NC, num_subcores=_NS)
    k = functools.partial(
        pl.kernel,
        out_type=jax.ShapeDtypeStruct((n, x.shape[1]), jnp.float32),
        mesh=mesh,
        scratch_types=[
            pltpu.VMEM((rpw,), jnp.int32),
            pltpu.VMEM((_GCH, x.shape[1]), jnp.float32),
            pltpu.SemaphoreType.DMA,
        ],
    )(functools.partial(_sc_gather_body, rpw))
    return k(x, idx)


def _gemm_body(be_ref, xs_ref, g_ref, w1_ref, b1_ref, w2_ref, b2_ref,
               w3_ref, b3_ref, out_ref):
    x = xs_ref[...].astype(jnp.bfloat16)
    h1 = jnp.maximum(
        jnp.dot(x, w1_ref[...].astype(jnp.bfloat16),
                preferred_element_type=jnp.float32)
        + b1_ref[...], 0.0).astype(jnp.bfloat16)
    h2 = jnp.maximum(
        jnp.dot(h1, w2_ref[...].astype(jnp.bfloat16),
                preferred_element_type=jnp.float32)
        + b2_ref[...], 0.0).astype(jnp.bfloat16)
    y = (jnp.dot(h2, w3_ref[...].astype(jnp.bfloat16),
                 preferred_element_type=jnp.float32)
         + b3_ref[...])
    out_ref[...] = y * g_ref[...]


def _grouped_mlp(xs, gs, block_expert, w1, b1, w2, b2, w3, b3):
    grid_spec = pltpu.PrefetchScalarGridSpec(
        num_scalar_prefetch=1,
        grid=(_NBLK,),
        in_specs=[
            pl.BlockSpec((_BB, _D), lambda j, be: (j, 0)),
            pl.BlockSpec((_BB, 1), lambda j, be: (j, 0)),
            pl.BlockSpec((_D, _H), lambda j, be: (0, be[j])),
            pl.BlockSpec((1, _H), lambda j, be: (0, be[j])),
            pl.BlockSpec((_H, _H), lambda j, be: (0, be[j])),
            pl.BlockSpec((1, _H), lambda j, be: (0, be[j])),
            pl.BlockSpec((_H, _O), lambda j, be: (0, be[j])),
            pl.BlockSpec((1, _O), lambda j, be: (0, be[j])),
        ],
        out_specs=pl.BlockSpec((_BB, _O), lambda j, be: (j, 0)),
    )
    return pl.pallas_call(
        _gemm_body,
        grid_spec=grid_spec,
        out_shape=jax.ShapeDtypeStruct((_L, _O), jnp.float32),
        compiler_params=pltpu.CompilerParams(
            dimension_semantics=("arbitrary",)),
    )(block_expert, xs, gs, w1, b1, w2, b2, w3, b3)


def _add_body(a_ref, b_ref, out_ref):
    out_ref[...] = a_ref[...] + b_ref[...]


def _sc_combine(ys, d0, d1):
    yg = _sc_gather(ys, jnp.concatenate([d0, d1]))
    tb = 256
    return pl.pallas_call(
        _add_body,
        grid=(_T // tb,),
        in_specs=[
            pl.BlockSpec((tb, _O), lambda i: (i, 0)),
            pl.BlockSpec((tb, _O), lambda i: (i + _T // tb, 0)),
        ],
        out_specs=pl.BlockSpec((tb, _O), lambda i: (i, 0)),
        out_shape=jax.ShapeDtypeStruct((_T, _O), jnp.float32),
        compiler_params=pltpu.CompilerParams(
            dimension_semantics=("arbitrary",)),
    )(yg, yg)


def kernel(x, gate_w, gate_b, gate_out_w, gate_out_b,
           mlp_w1, mlp_b1, mlp_w2, mlp_b2, mlp_w3, mlp_b3):
    i0, i1, g0, g1 = _gate_topk(x, gate_w, gate_b, gate_out_w, gate_out_b)
    top_idx = jnp.concatenate([i0, i1], axis=1)
    gates = jnp.concatenate([g0, g1], axis=1)

    ef = top_idx.reshape(-1)
    oh = (ef[:, None] == jnp.arange(_E, dtype=jnp.int32)[None, :])
    oh = oh.astype(jnp.int32)
    pos = jnp.cumsum(oh, axis=0) - oh
    pos_e = jnp.take_along_axis(pos, ef[:, None], axis=1)[:, 0]
    counts = jnp.sum(oh, axis=0)
    padded = ((counts + _BB - 1) // _BB) * _BB
    cum = jnp.cumsum(padded)
    start = cum - padded
    dest = start[ef] + pos_e
    tok = (jnp.arange(_T * _K, dtype=jnp.int32) // _K)
    tok_sorted = jnp.zeros((_L,), jnp.int32).at[dest].set(tok)
    gate_sorted = (jnp.zeros((_L,), jnp.float32)
                   .at[dest].set(gates.reshape(-1))).reshape(_L, 1)
    block_expert = jnp.minimum(
        jnp.searchsorted(cum, jnp.arange(_NBLK, dtype=jnp.int32) * _BB,
                         side="right"),
        _E - 1).astype(jnp.int32)
    dest2 = dest.reshape(_T, _K).astype(jnp.int32)

    xs = _sc_gather(x, tok_sorted)

    w1 = mlp_w1.reshape(_D, _E * _H)
    w2 = mlp_w2.reshape(_H, _E * _H)
    w3 = mlp_w3.reshape(_H, _E * _O)
    b1 = mlp_b1.reshape(1, _E * _H)
    b2 = mlp_b2.reshape(1, _E * _H)
    b3 = mlp_b3.reshape(1, _E * _O)
    ys = _grouped_mlp(xs, gate_sorted, block_expert,
                      w1, b1, w2, b2, w3, b3)

    return _sc_combine(ys, dest2[:, 0], dest2[:, 1])


# speedup vs baseline: 1.4776x; 1.4776x over previous
import functools

import jax
import jax.numpy as jnp
from jax import lax
from jax.experimental import pallas as pl
from jax.experimental.pallas import tpu as pltpu
from jax.experimental.pallas import tpu_sc as plsc

_T, _D, _H, _E, _O, _K = 2048, 1024, 1024, 8, 1024, 2

_BB = 256
_L = _T * _K + _E * _BB
_NBLK = _L // _BB

_NC, _NS = 2, 16
_NW = _NC * _NS


def _gate_body(x_ref, gw_ref, gb_ref, gow_ref, gob_ref,
               i0_ref, i1_ref, g0_ref, g1_ref):
    x = x_ref[...]
    h = jnp.maximum(
        jnp.dot(x, gw_ref[...], preferred_element_type=jnp.float32)
        + gb_ref[...], 0.0)
    logits = (jnp.dot(h, gow_ref[...], preferred_element_type=jnp.float32)
              + gob_ref[...])
    cols = lax.broadcasted_iota(jnp.int32, logits.shape, 1)
    v0 = jnp.max(logits, axis=1, keepdims=True)
    i0 = jnp.min(jnp.where(logits == v0, cols, _E), axis=1, keepdims=True)
    masked = jnp.where(cols == i0, -jnp.inf, logits)
    v1 = jnp.max(masked, axis=1, keepdims=True)
    i1 = jnp.min(jnp.where(masked == v1, cols, _E), axis=1, keepdims=True)
    e1 = jnp.exp(v1 - v0)
    g0 = 1.0 / (1.0 + e1)
    i0_ref[...] = i0
    i1_ref[...] = i1
    g0_ref[...] = g0
    g1_ref[...] = 1.0 - g0


def _gate_topk(x, gate_w, gate_b, gate_out_w, gate_out_b):
    tb = 512
    grid = (_T // tb,)
    out_shape = [
        jax.ShapeDtypeStruct((_T, 1), jnp.int32),
        jax.ShapeDtypeStruct((_T, 1), jnp.int32),
        jax.ShapeDtypeStruct((_T, 1), jnp.float32),
        jax.ShapeDtypeStruct((_T, 1), jnp.float32),
    ]
    tspec = lambda: pl.BlockSpec((tb, 1), lambda i: (i, 0))
    return pl.pallas_call(
        _gate_body,
        grid=grid,
        in_specs=[
            pl.BlockSpec((tb, _D), lambda i: (i, 0)),
            pl.BlockSpec((_D, _H), lambda i: (0, 0)),
            pl.BlockSpec((1, _H), lambda i: (0, 0)),
            pl.BlockSpec((_H, _E), lambda i: (0, 0)),
            pl.BlockSpec((1, _E), lambda i: (0, 0)),
        ],
        out_specs=[tspec(), tspec(), tspec(), tspec()],
        out_shape=out_shape,
        compiler_params=pltpu.CompilerParams(
            dimension_semantics=("arbitrary",)),
    )(x, gate_w, gate_b.reshape(1, _H), gate_out_w, gate_out_b.reshape(1, _E))


_XPW = _T // _NW


def _sc_dispatch_body(x_hbm, d0_hbm, d1_hbm, out_hbm, d0_v, d1_v, rows_v,
                      s0, s1):
    wid = lax.axis_index("s") * _NC + lax.axis_index("c")
    base = wid * _XPW
    pltpu.sync_copy(d0_hbm.at[pl.ds(base, _XPW)], d0_v)
    pltpu.sync_copy(d1_hbm.at[pl.ds(base, _XPW)], d1_v)
    pltpu.sync_copy(x_hbm.at[pl.ds(base, _XPW)], rows_v)
    c0 = pltpu.async_copy(rows_v, out_hbm.at[d0_v], s0)
    c1 = pltpu.async_copy(rows_v, out_hbm.at[d1_v], s1)
    c0.wait()
    c1.wait()


def _sc_dispatch(x, d0, d1):
    mesh = plsc.VectorSubcoreMesh(core_axis_name="c", subcore_axis_name="s",
                                  num_cores=_NC, num_subcores=_NS)
    k = functools.partial(
        pl.kernel,
        out_type=jax.ShapeDtypeStruct((_L, _D), jnp.float32),
        mesh=mesh,
        scratch_types=[
            pltpu.VMEM((_XPW,), jnp.int32),
            pltpu.VMEM((_XPW,), jnp.int32),
            pltpu.VMEM((_XPW, _D), jnp.float32),
            pltpu.SemaphoreType.DMA,
            pltpu.SemaphoreType.DMA,
        ],
    )(_sc_dispatch_body)
    return k(x, d0, d1)


def _gemm_body(be_ref, xs_ref, w1_ref, b1_ref, w2_ref, b2_ref,
               w3_ref, b3_ref, out_ref):
    @pl.when(pl.program_id(0) < be_ref[_NBLK])
    def _():
        _gemm_compute(xs_ref, w1_ref, b1_ref, w2_ref, b2_ref,
                      w3_ref, b3_ref, out_ref)


def _gemm_compute(xs_ref, w1_ref, b1_ref, w2_ref, b2_ref,
                  w3_ref, b3_ref, out_ref):
    x = xs_ref[...].astype(jnp.bfloat16)
    h1 = jnp.maximum(
        jnp.dot(x, w1_ref[...].astype(jnp.bfloat16),
                preferred_element_type=jnp.float32)
        + b1_ref[...], 0.0).astype(jnp.bfloat16)
    h2 = jnp.maximum(
        jnp.dot(h1, w2_ref[...].astype(jnp.bfloat16),
                preferred_element_type=jnp.float32)
        + b2_ref[...], 0.0).astype(jnp.bfloat16)
    y = (jnp.dot(h2, w3_ref[...].astype(jnp.bfloat16),
                 preferred_element_type=jnp.float32)
         + b3_ref[...])
    out_ref[...] = y


def _grouped_mlp(xs, block_expert, w1, b1, w2, b2, w3, b3):
    grid_spec = pltpu.PrefetchScalarGridSpec(
        num_scalar_prefetch=1,
        grid=(_NBLK,),
        in_specs=[
            pl.BlockSpec((_BB, _D), lambda j, be: (j, 0)),
            pl.BlockSpec((_D, _H), lambda j, be: (0, be[j])),
            pl.BlockSpec((1, _H), lambda j, be: (0, be[j])),
            pl.BlockSpec((_H, _H), lambda j, be: (0, be[j])),
            pl.BlockSpec((1, _H), lambda j, be: (0, be[j])),
            pl.BlockSpec((_H, _O), lambda j, be: (0, be[j])),
            pl.BlockSpec((1, _O), lambda j, be: (0, be[j])),
        ],
        out_specs=pl.BlockSpec((_BB, _O), lambda j, be: (j, 0)),
    )
    return pl.pallas_call(
        _gemm_body,
        grid_spec=grid_spec,
        out_shape=jax.ShapeDtypeStruct((_L, _O), jnp.float32),
        compiler_params=pltpu.CompilerParams(
            dimension_semantics=("arbitrary",)),
    )(block_expert, xs, w1, b1, w2, b2, w3, b3)


_CPW = _T // _NW
_CC = 32


def _sc_combine_body(y_hbm, d0_hbm, d1_hbm, g0_hbm, g1_hbm, out_hbm,
                     d0_v, d1_v, g0_v, g1_v, ya_v, yb_v, s0, s1):
    wid = lax.axis_index("s") * _NC + lax.axis_index("c")
    base = wid * _CPW
    pltpu.sync_copy(d0_hbm.at[pl.ds(base, _CPW)], d0_v)
    pltpu.sync_copy(d1_hbm.at[pl.ds(base, _CPW)], d1_v)
    pltpu.sync_copy(g0_hbm.at[pl.ds(base, _CPW)], g0_v)
    pltpu.sync_copy(g1_hbm.at[pl.ds(base, _CPW)], g1_v)
    for c in range(_CPW // _CC):
        c0 = pltpu.async_copy(
            y_hbm.at[d0_v.at[pl.ds(c * _CC, _CC)]], ya_v, s0)
        c1 = pltpu.async_copy(
            y_hbm.at[d1_v.at[pl.ds(c * _CC, _CC)]], yb_v, s1)
        c0.wait()
        c1.wait()

        def _row(r, _):
            t = c * _CC + r
            av = g0_v[t, pl.ds(0, 16)]
            bv = g1_v[t, pl.ds(0, 16)]
            for q in range(_O // 16):
                ya_v[r, pl.ds(q * 16, 16)] = (
                    av * ya_v[r, pl.ds(q * 16, 16)]
                    + bv * yb_v[r, pl.ds(q * 16, 16)])
            return 0

        lax.fori_loop(0, _CC, _row, 0)
        pltpu.sync_copy(ya_v, out_hbm.at[pl.ds(base + c * _CC, _CC)])


def _sc_combine(ys, d0, d1, g0, g1):
    g0b = jnp.broadcast_to(g0.reshape(_T, 1), (_T, 16))
    g1b = jnp.broadcast_to(g1.reshape(_T, 1), (_T, 16))
    mesh = plsc.VectorSubcoreMesh(core_axis_name="c", subcore_axis_name="s",
                                  num_cores=_NC, num_subcores=_NS)
    k = functools.partial(
        pl.kernel,
        out_type=jax.ShapeDtypeStruct((_T, _O), jnp.float32),
        mesh=mesh,
        scratch_types=[
            pltpu.VMEM((_CPW,), jnp.int32),
            pltpu.VMEM((_CPW,), jnp.int32),
            pltpu.VMEM((_CPW, 16), jnp.float32),
            pltpu.VMEM((_CPW, 16), jnp.float32),
            pltpu.VMEM((_CC, _O), jnp.float32),
            pltpu.VMEM((_CC, _O), jnp.float32),
            pltpu.SemaphoreType.DMA,
            pltpu.SemaphoreType.DMA,
        ],
    )(_sc_combine_body)
    return k(ys, d0, d1, g0b, g1b)


def kernel(x, gate_w, gate_b, gate_out_w, gate_out_b,
           mlp_w1, mlp_b1, mlp_w2, mlp_b2, mlp_w3, mlp_b3):
    i0, i1, g0, g1 = _gate_topk(x, gate_w, gate_b, gate_out_w, gate_out_b)
    top_idx = jnp.concatenate([i0, i1], axis=1)

    ef = top_idx.reshape(-1)
    oh = (ef[:, None] == jnp.arange(_E, dtype=jnp.int32)[None, :])
    oh = oh.astype(jnp.int32)
    pos = jnp.cumsum(oh, axis=0) - oh
    pos_e = jnp.take_along_axis(pos, ef[:, None], axis=1)[:, 0]
    counts = jnp.sum(oh, axis=0)
    padded = ((counts + _BB - 1) // _BB) * _BB
    cum = jnp.cumsum(padded)
    start = cum - padded
    dest = start[ef] + pos_e
    block_expert = jnp.minimum(
        jnp.searchsorted(cum, jnp.arange(_NBLK, dtype=jnp.int32) * _BB,
                         side="right"),
        _E - 1).astype(jnp.int32)
    block_expert = jnp.concatenate(
        [block_expert, (cum[_E - 1] // _BB).astype(jnp.int32)[None]])
    dest2 = dest.reshape(_T, _K).astype(jnp.int32)
    d0 = dest2[:, 0]
    d1 = dest2[:, 1]

    xs = _sc_dispatch(x, d0, d1)

    w1 = mlp_w1.reshape(_D, _E * _H)
    w2 = mlp_w2.reshape(_H, _E * _H)
    w3 = mlp_w3.reshape(_H, _E * _O)
    b1 = mlp_b1.reshape(1, _E * _H)
    b2 = mlp_b2.reshape(1, _E * _H)
    b3 = mlp_b3.reshape(1, _E * _O)
    ys = _grouped_mlp(xs, block_expert,
                      w1, b1, w2, b2, w3, b3)

    return _sc_combine(ys, d0, d1, g0[:, 0], g1[:, 0])
